# baseline (speedup 1.0000x reference)
import functools

import jax
import jax.numpy as jnp
from jax import lax
from jax.experimental import pallas as pl
from jax.experimental.pallas import tpu as pltpu
from jax.experimental.pallas import tpu_sc as plsc

K = 512
D = 64
BETA = 0.25
DECAY = 0.99
EPS = 1e-05

N = 2
HW = 224
PLANE = HW * HW
PAD = 256
PLANEP = PLANE + 2 * PAD
LT = 3584
NT = PLANE // LT
ROWS = N * 64 * PLANE // D
RT = 1024
NRT = ROWS // RT
NPLANES = N * 64
PROWS = PLANE // D
PROWSP = PROWS + 8

_EMB_N = float(N * 64 * HW * HW)
_REC_N = float(N * 3 * HW * HW)

_NC = 2
_NS = 16
_NW = _NC * _NS
_WPL = NPLANES // _NW
_CH = 88
_NCH = PROWSP // _CH


def _shift_slices(ref, j, lt):
    lanes = lax.broadcasted_iota(jnp.int32, (1, lt), 1)
    w = lanes % HW
    mask_l = (w != 0).astype(jnp.float32)
    mask_r = (w != HW - 1).astype(jnp.float32)
    start = pl.multiple_of(j * lt, 128)
    wide = ref[0, :, pl.ds(start, lt + 2 * PAD)]
    parts = []
    for dy in (-1, 0, 1):
        for dx in (-1, 0, 1):
            s = dy * HW + dx
            sl = lax.slice_in_dim(wide, PAD + s, PAD + s + lt, axis=1)
            if dx == -1:
                sl = sl * mask_l
            elif dx == 1:
                sl = sl * mask_r
            parts.append(sl)
    return parts


def _enc_body(imgp_ref, w_ref, b_ref, x_ref):
    j = pl.program_id(1)
    p = jnp.concatenate(_shift_slices(imgp_ref, j, LT), axis=0)
    y = lax.dot_general(w_ref[...], p, (((1,), (0,)), ((), ())),
                        preferred_element_type=jnp.float32)
    x_ref[0] = y + b_ref[...]


def _vq_body(f_ref, cb_ref, ecs_ref, eis_ref, near_ref, cbn_ref,
             ohs_acc, is_acc):
    i = pl.program_id(0)

    @pl.when(i == 0)
    def _():
        ohs_acc[...] = jnp.zeros_like(ohs_acc)
        is_acc[...] = jnp.zeros_like(is_acc)

    f = f_ref[...]
    cb = cb_ref[...]
    dot = lax.dot_general(f, cb, (((1,), (0,)), ((), ())),
                          preferred_element_type=jnp.float32)
    dist = jnp.sum(cb * cb, axis=0, keepdims=True) - 2.0 * dot
    m = jnp.min(dist, axis=1, keepdims=True)
    iota_k = lax.broadcasted_iota(jnp.int32, (RT, K), 1)
    near = jnp.min(jnp.where(dist == m, iota_k, K), axis=1, keepdims=True)
    near_ref[0] = near
    oh = (iota_k == near).astype(jnp.float32)
    ohs_acc[...] += jnp.sum(oh, axis=0, keepdims=True)
    is_acc[...] += lax.dot_general(f, oh, (((0,), (0,)), ((), ())),
                                   preferred_element_type=jnp.float32)

    @pl.when(i == NRT - 1)
    def _():
        ecs_new = ecs_ref[...] * DECAY + (1.0 - DECAY) * ohs_acc[...]
        eis_new = eis_ref[...] * DECAY + (1.0 - DECAY) * is_acc[...]
        n = jnp.sum(ecs_new)
        cs = (ecs_new + EPS) / (n + K * EPS) * n
        cbn_ref[...] = eis_new / cs


def _sc_gather_body(cbt_hbm, idx_hbm, out_hbm, idx_v, rows_v, sem):
    wid = lax.axis_index("s") * _NC + lax.axis_index("c")
    pltpu.sync_copy(idx_hbm.at[wid], idx_v)
    base = wid * _WPL * PROWSP
    for p in range(_WPL):
        for c in range(_NCH):
            pltpu.async_copy(cbt_hbm.at[idx_v.at[p * _NCH + c]],
                             rows_v.at[pl.ds(c * _CH, _CH)], sem)
        for c in range(_NCH):
            pltpu.make_async_copy(
                cbt_hbm.at[idx_v.at[p * _NCH + c]],
                rows_v.at[pl.ds(c * _CH, _CH)], sem).wait()
        pltpu.sync_copy(rows_v, out_hbm.at[pl.ds(base + p * PROWSP, PROWSP)])


def _sc_gather(cbt, idx):
    run = functools.partial(
        pl.kernel,
        mesh=plsc.VectorSubcoreMesh(core_axis_name="c", subcore_axis_name="s"),
        compiler_params=pltpu.CompilerParams(use_tc_tiling_on_sc=False),
        out_type=jax.ShapeDtypeStruct((NPLANES * PROWSP, D), jnp.float32),
        scratch_types=[
            pltpu.VMEM((_WPL * _NCH, _CH), jnp.int32),
            pltpu.VMEM((PROWSP, D), jnp.float32),
            pltpu.SemaphoreType.DMA,
        ],
    )(_sc_gather_body)
    return run(cbt, idx)


def _dec_body(qp_ref, x_ref, img_ref, wd_ref, db_ref, pred_ref, loss_ref,
              acc_ref):
    n = pl.program_id(0)
    j = pl.program_id(1)
    step = n * NT + j

    @pl.when(step == 0)
    def _():
        acc_ref[0] = 0.0
        acc_ref[1] = 0.0

    start = pl.multiple_of(j * LT, 128)
    wide = qp_ref[0, :, pl.ds(start, LT + 2 * PAD)]
    qc = lax.slice_in_dim(wide, PAD, PAD + LT, axis=1)
    acc_ref[0] += jnp.sum((qc - x_ref[0]) ** 2)

    g = lax.dot_general(wd_ref[...], wide, (((1,), (0,)), ((), ())),
                        preferred_element_type=jnp.float32)
    lanes = lax.broadcasted_iota(jnp.int32, (1, LT), 1)
    w = lanes % HW
    mask_l = (w != 0).astype(jnp.float32)
    mask_r = (w != HW - 1).astype(jnp.float32)
    y = db_ref[...] * jnp.ones((8, LT), jnp.float32)
    t = 0
    for dy in (-1, 0, 1):
        for dx in (-1, 0, 1):
            s = dy * HW + dx
            part = lax.slice(g, (8 * t, PAD + s), (8 * t + 8, PAD + s + LT))
            if dx == -1:
                part = part * mask_l
            elif dx == 1:
                part = part * mask_r
            y = y + part
            t += 1
    pred_ref[0] = y
    acc_ref[1] += jnp.sum((y - img_ref[0]) ** 2)

    @pl.when(step == N * NT - 1)
    def _():
        loss = acc_ref[1] / _REC_N + BETA * acc_ref[0] / _EMB_N
        loss_ref[...] = jnp.full((1, 128), loss, dtype=jnp.float32)


def kernel(images, enc_w, enc_b, dec_w, dec_b, codebook,
           ema_cluster_size, ema_input_sum):
    f32 = jnp.float32
    imgs_flat = images.reshape(N, 3, PLANE)
    imgp = jnp.pad(imgs_flat, ((0, 0), (0, 0), (PAD, PAD)))

    w_enc = enc_w.transpose(0, 2, 3, 1).reshape(64, 27)
    w_dec = jnp.zeros((9, 8, 64), f32).at[:, :3].set(
        dec_w.transpose(2, 3, 0, 1).reshape(9, 3, 64)).reshape(72, 64)
    db8 = jnp.zeros((8, 1), f32).at[:3, 0].set(dec_b)

    x = pl.pallas_call(
        _enc_body,
        grid=(N, NT),
        in_specs=[
            pl.BlockSpec((1, 3, PLANEP), lambda n, j: (n, 0, 0)),
            pl.BlockSpec((64, 27), lambda n, j: (0, 0)),
            pl.BlockSpec((64, 1), lambda n, j: (0, 0)),
        ],
        out_specs=pl.BlockSpec((1, 64, LT), lambda n, j: (n, 0, j)),
        out_shape=jax.ShapeDtypeStruct((N, 64, PLANE), f32),
        compiler_params=pltpu.CompilerParams(
            dimension_semantics=("arbitrary", "arbitrary")),
    )(imgp, w_enc, enc_b.reshape(64, 1))

    flat = x.reshape(ROWS, D)
    near, cbn = pl.pallas_call(
        _vq_body,
        grid=(NRT,),
        in_specs=[
            pl.BlockSpec((RT, D), lambda i: (i, 0)),
            pl.BlockSpec((D, K), lambda i: (0, 0)),
            pl.BlockSpec((1, K), lambda i: (0, 0)),
            pl.BlockSpec((D, K), lambda i: (0, 0)),
        ],
        out_specs=[
            pl.BlockSpec((1, RT, 1), lambda i: (i, 0, 0)),
            pl.BlockSpec((D, K), lambda i: (0, 0)),
        ],
        out_shape=[
            jax.ShapeDtypeStruct((NRT, RT, 1), jnp.int32),
            jax.ShapeDtypeStruct((D, K), f32),
        ],
        scratch_shapes=[
            pltpu.VMEM((1, K), f32),
            pltpu.VMEM((D, K), f32),
        ],
        compiler_params=pltpu.CompilerParams(
            dimension_semantics=("arbitrary",)),
    )(flat, codebook, ema_cluster_size.reshape(1, K), ema_input_sum)

    near_pl = jnp.pad(near.reshape(NPLANES, PROWS), ((0, 0), (4, 4)),
                      constant_values=K)
    idx = near_pl.reshape(_NW, _WPL * _NCH, _CH)

    cbt = jnp.zeros((K + 8, D), f32).at[:K].set(cbn.T)
    qp = _sc_gather(cbt, idx).reshape(N, 64, PLANEP)

    img8 = jnp.zeros((N, 8, PLANE), f32).at[:, :3].set(imgs_flat)
    pred8, loss_v = pl.pallas_call(
        _dec_body,
        grid=(N, NT),
        in_specs=[
            pl.BlockSpec((1, 64, PLANEP), lambda n, j: (n, 0, 0)),
            pl.BlockSpec((1, 64, LT), lambda n, j: (n, 0, j)),
            pl.BlockSpec((1, 8, LT), lambda n, j: (n, 0, j)),
            pl.BlockSpec((72, 64), lambda n, j: (0, 0)),
            pl.BlockSpec((8, 1), lambda n, j: (0, 0)),
        ],
        out_specs=[
            pl.BlockSpec((1, 8, LT), lambda n, j: (n, 0, j)),
            pl.BlockSpec((1, 128), lambda n, j: (0, 0)),
        ],
        out_shape=[
            jax.ShapeDtypeStruct((N, 8, PLANE), f32),
            jax.ShapeDtypeStruct((1, 128), f32),
        ],
        scratch_shapes=[pltpu.SMEM((2,), f32)],
        compiler_params=pltpu.CompilerParams(
            dimension_semantics=("arbitrary", "arbitrary")),
    )(qp, x, img8, w_dec, db8)

    pred_images = pred8[:, :3].reshape(N, 3, HW, HW)
    loss = loss_v[0, 0]
    return pred_images, loss


# speedup vs baseline: 2.0329x; 2.0329x over previous
import functools

import jax
import jax.numpy as jnp
from jax import lax
from jax.experimental import pallas as pl
from jax.experimental.pallas import tpu as pltpu
from jax.experimental.pallas import tpu_sc as plsc

K = 512
D = 64
BETA = 0.25
DECAY = 0.99
EPS = 1e-05

N = 2
HW = 224
PLANE = HW * HW
PAD = 256
PLANEP = PLANE + 2 * PAD
LT = 3584
NT = PLANE // LT
ROWS = N * 64 * PLANE // D
RT = 1024
NRT = ROWS // RT
NPLANES = N * 64
PROWS = PLANE // D
PROWSP = PROWS + 8

_EMB_N = float(N * 64 * HW * HW)
_REC_N = float(N * 3 * HW * HW)

_NC = 2
_NS = 16
_NW = _NC * _NS
_WPL = NPLANES // _NW
_CH = 88
_NCH = PROWSP // _CH


def _shift_slices(ref, j, lt):
    lanes = lax.broadcasted_iota(jnp.int32, (1, lt), 1)
    w = lanes % HW
    mask_l = (w != 0).astype(jnp.float32)
    mask_r = (w != HW - 1).astype(jnp.float32)
    start = pl.multiple_of(j * lt, 128)
    wide = ref[0, :, pl.ds(start, lt + 2 * PAD)]
    parts = []
    for dy in (-1, 0, 1):
        for dx in (-1, 0, 1):
            s = dy * HW + dx
            sl = lax.slice_in_dim(wide, PAD + s, PAD + s + lt, axis=1)
            if dx == -1:
                sl = sl * mask_l
            elif dx == 1:
                sl = sl * mask_r
            parts.append(sl)
    return parts


def _enc_body(imgp_ref, w_ref, b_ref, x_ref):
    j = pl.program_id(1)
    p = jnp.concatenate(_shift_slices(imgp_ref, j, LT), axis=0)
    y = lax.dot_general(w_ref[...], p, (((1,), (0,)), ((), ())),
                        preferred_element_type=jnp.float32)
    x_ref[0] = y + b_ref[...]


def _vq_body(f_ref, cb_ref, ecs_ref, eis_ref, near_ref, cbn_ref,
             ohs_acc, is_acc):
    i = pl.program_id(0)

    @pl.when(i == 0)
    def _():
        ohs_acc[...] = jnp.zeros_like(ohs_acc)
        is_acc[...] = jnp.zeros_like(is_acc)

    f = f_ref[...]
    cb = cb_ref[...]
    dot = lax.dot_general(f, cb, (((1,), (0,)), ((), ())),
                          preferred_element_type=jnp.float32)
    dist = jnp.sum(cb * cb, axis=0, keepdims=True) - 2.0 * dot
    m = jnp.min(dist, axis=1, keepdims=True)
    iota_k = lax.broadcasted_iota(jnp.int32, (RT, K), 1)
    near = jnp.min(jnp.where(dist == m, iota_k, K), axis=1, keepdims=True)
    near_ref[0] = near
    oh = (iota_k == near).astype(jnp.float32)
    ohs_acc[...] += jnp.sum(oh, axis=0, keepdims=True)
    is_acc[...] += lax.dot_general(f, oh, (((0,), (0,)), ((), ())),
                                   preferred_element_type=jnp.float32)

    @pl.when(i == NRT - 1)
    def _():
        ecs_new = ecs_ref[...] * DECAY + (1.0 - DECAY) * ohs_acc[...]
        eis_new = eis_ref[...] * DECAY + (1.0 - DECAY) * is_acc[...]
        n = jnp.sum(ecs_new)
        cs = (ecs_new + EPS) / (n + K * EPS) * n
        cbn_ref[...] = eis_new / cs


_USE_SC_GATHER = False


def _gather_body(near_ref, cbn_ref, q_ref):
    near = near_ref[0]
    iota_k = lax.broadcasted_iota(jnp.int32, (PROWSP, K), 1)
    oh = (iota_k == near).astype(jnp.float32)
    q_ref[0] = lax.dot_general(oh, cbn_ref[...], (((1,), (1,)), ((), ())),
                               preferred_element_type=jnp.float32)


def _sc_gather_body(cbt_hbm, idx_hbm, out_hbm, idx_v, rows_v, sem):
    wid = lax.axis_index("s") * _NC + lax.axis_index("c")
    pltpu.sync_copy(idx_hbm.at[wid], idx_v)
    base = wid * _WPL * PROWSP
    for p in range(_WPL):
        for c in range(_NCH):
            pltpu.async_copy(cbt_hbm.at[idx_v.at[p * _NCH + c]],
                             rows_v.at[pl.ds(c * _CH, _CH)], sem)
        for c in range(_NCH):
            pltpu.make_async_copy(
                cbt_hbm.at[idx_v.at[p * _NCH + c]],
                rows_v.at[pl.ds(c * _CH, _CH)], sem).wait()
        pltpu.sync_copy(rows_v, out_hbm.at[pl.ds(base + p * PROWSP, PROWSP)])


def _sc_gather(cbt, idx):
    run = functools.partial(
        pl.kernel,
        mesh=plsc.VectorSubcoreMesh(core_axis_name="c", subcore_axis_name="s"),
        compiler_params=pltpu.CompilerParams(use_tc_tiling_on_sc=False),
        out_type=jax.ShapeDtypeStruct((NPLANES * PROWSP, D), jnp.float32),
        scratch_types=[
            pltpu.VMEM((_WPL * _NCH, _CH), jnp.int32),
            pltpu.VMEM((PROWSP, D), jnp.float32),
            pltpu.SemaphoreType.DMA,
        ],
    )(_sc_gather_body)
    return run(cbt, idx)


def _dec_body(qp_ref, x_ref, img_ref, wd_ref, db_ref, pred_ref, loss_ref,
              acc_ref):
    n = pl.program_id(0)
    j = pl.program_id(1)
    step = n * NT + j

    @pl.when(step == 0)
    def _():
        acc_ref[0] = 0.0
        acc_ref[1] = 0.0

    start = pl.multiple_of(j * LT, 128)
    wide = qp_ref[0, :, pl.ds(start, LT + 2 * PAD)]
    qc = lax.slice_in_dim(wide, PAD, PAD + LT, axis=1)
    acc_ref[0] += jnp.sum((qc - x_ref[0]) ** 2)

    g = lax.dot_general(wd_ref[...], wide, (((1,), (0,)), ((), ())),
                        preferred_element_type=jnp.float32)
    lanes = lax.broadcasted_iota(jnp.int32, (1, LT), 1)
    w = lanes % HW
    mask_l = (w != 0).astype(jnp.float32)
    mask_r = (w != HW - 1).astype(jnp.float32)
    y = db_ref[...] * jnp.ones((8, LT), jnp.float32)
    t = 0
    for dy in (-1, 0, 1):
        for dx in (-1, 0, 1):
            s = dy * HW + dx
            part = lax.slice(g, (8 * t, PAD + s), (8 * t + 8, PAD + s + LT))
            if dx == -1:
                part = part * mask_l
            elif dx == 1:
                part = part * mask_r
            y = y + part
            t += 1
    pred_ref[0] = y
    acc_ref[1] += jnp.sum((y - img_ref[0]) ** 2)

    @pl.when(step == N * NT - 1)
    def _():
        loss = acc_ref[1] / _REC_N + BETA * acc_ref[0] / _EMB_N
        loss_ref[...] = jnp.full((1, 128), loss, dtype=jnp.float32)


def kernel(images, enc_w, enc_b, dec_w, dec_b, codebook,
           ema_cluster_size, ema_input_sum):
    f32 = jnp.float32
    imgs_flat = images.reshape(N, 3, PLANE)
    imgp = jnp.pad(imgs_flat, ((0, 0), (0, 0), (PAD, PAD)))

    w_enc = enc_w.transpose(0, 2, 3, 1).reshape(64, 27)
    w_dec = jnp.zeros((9, 8, 64), f32).at[:, :3].set(
        dec_w.transpose(2, 3, 0, 1).reshape(9, 3, 64)).reshape(72, 64)
    db8 = jnp.zeros((8, 1), f32).at[:3, 0].set(dec_b)

    x = pl.pallas_call(
        _enc_body,
        grid=(N, NT),
        in_specs=[
            pl.BlockSpec((1, 3, PLANEP), lambda n, j: (n, 0, 0)),
            pl.BlockSpec((64, 27), lambda n, j: (0, 0)),
            pl.BlockSpec((64, 1), lambda n, j: (0, 0)),
        ],
        out_specs=pl.BlockSpec((1, 64, LT), lambda n, j: (n, 0, j)),
        out_shape=jax.ShapeDtypeStruct((N, 64, PLANE), f32),
        compiler_params=pltpu.CompilerParams(
            dimension_semantics=("arbitrary", "arbitrary")),
    )(imgp, w_enc, enc_b.reshape(64, 1))

    flat = x.reshape(ROWS, D)
    near, cbn = pl.pallas_call(
        _vq_body,
        grid=(NRT,),
        in_specs=[
            pl.BlockSpec((RT, D), lambda i: (i, 0)),
            pl.BlockSpec((D, K), lambda i: (0, 0)),
            pl.BlockSpec((1, K), lambda i: (0, 0)),
            pl.BlockSpec((D, K), lambda i: (0, 0)),
        ],
        out_specs=[
            pl.BlockSpec((1, RT, 1), lambda i: (i, 0, 0)),
            pl.BlockSpec((D, K), lambda i: (0, 0)),
        ],
        out_shape=[
            jax.ShapeDtypeStruct((NRT, RT, 1), jnp.int32),
            jax.ShapeDtypeStruct((D, K), f32),
        ],
        scratch_shapes=[
            pltpu.VMEM((1, K), f32),
            pltpu.VMEM((D, K), f32),
        ],
        compiler_params=pltpu.CompilerParams(
            dimension_semantics=("arbitrary",)),
    )(flat, codebook, ema_cluster_size.reshape(1, K), ema_input_sum)

    near_pl = jnp.pad(near.reshape(NPLANES, PROWS), ((0, 0), (4, 4)),
                      constant_values=K)
    idx = near_pl.reshape(_NW, _WPL * _NCH, _CH)

    if _USE_SC_GATHER:
        cbt = jnp.zeros((K + 8, D), f32).at[:K].set(cbn.T)
        qp_rows = _sc_gather(cbt, idx)
    else:
        qp_rows = pl.pallas_call(
            _gather_body,
            grid=(NPLANES,),
            in_specs=[
                pl.BlockSpec((1, PROWSP, 1), lambda i: (i, 0, 0)),
                pl.BlockSpec((D, K), lambda i: (0, 0)),
            ],
            out_specs=pl.BlockSpec((1, PROWSP, D), lambda i: (i, 0, 0)),
            out_shape=jax.ShapeDtypeStruct((NPLANES, PROWSP, D), f32),
            compiler_params=pltpu.CompilerParams(
                dimension_semantics=("arbitrary",)),
        )(near_pl.reshape(NPLANES, PROWSP, 1), cbn)
    qp = qp_rows.reshape(N, 64, PLANEP)

    img8 = jnp.zeros((N, 8, PLANE), f32).at[:, :3].set(imgs_flat)
    pred8, loss_v = pl.pallas_call(
        _dec_body,
        grid=(N, NT),
        in_specs=[
            pl.BlockSpec((1, 64, PLANEP), lambda n, j: (n, 0, 0)),
            pl.BlockSpec((1, 64, LT), lambda n, j: (n, 0, j)),
            pl.BlockSpec((1, 8, LT), lambda n, j: (n, 0, j)),
            pl.BlockSpec((72, 64), lambda n, j: (0, 0)),
            pl.BlockSpec((8, 1), lambda n, j: (0, 0)),
        ],
        out_specs=[
            pl.BlockSpec((1, 8, LT), lambda n, j: (n, 0, j)),
            pl.BlockSpec((1, 128), lambda n, j: (0, 0)),
        ],
        out_shape=[
            jax.ShapeDtypeStruct((N, 8, PLANE), f32),
            jax.ShapeDtypeStruct((1, 128), f32),
        ],
        scratch_shapes=[pltpu.SMEM((2,), f32)],
        compiler_params=pltpu.CompilerParams(
            dimension_semantics=("arbitrary", "arbitrary")),
    )(qp, x, img8, w_dec, db8)

    pred_images = pred8[:, :3].reshape(N, 3, HW, HW)
    loss = loss_v[0, 0]
    return pred_images, loss
